# Initial kernel scaffold; baseline (speedup 1.0000x reference)
import jax
import jax.numpy as jnp
from jax.experimental import pallas as pl


def kernel(x, codebook, selector_centroids):
    raise NotImplementedError("write your pallas kernel here")


# speedup vs baseline: 7.2944x; 7.2944x over previous
import functools

import jax
import jax.numpy as jnp
from jax import lax
from jax.experimental import pallas as pl
from jax.experimental.pallas import tpu as pltpu
from jax.experimental.pallas import tpu_sc as plsc

D_VECTOR = 256
N_DATA = 4096
N_SUB = 8
D_SUB = D_VECTOR // N_SUB
N_CLUSTERS = 256
N_CB = 64

BLK = 256
N_BLOCKS = N_DATA // BLK
CHUNK = 128

F32 = jnp.float32
I32 = jnp.int32


def _prep_body(x_ref, c_ref, xt_ref, labels_ref, pos_ref, counts_ref,
               offsets_ref):
    x = x_ref[...]
    xt = x.T
    xt_ref[...] = xt
    cent = c_ref[...]

    x_sq = jnp.sum(xt * xt, axis=1, keepdims=True)
    c_sq = jnp.sum(cent * cent, axis=0, keepdims=True)
    xc = lax.dot_general(xt, cent, (((1,), (0,)), ((), ())),
                         preferred_element_type=F32)
    d2 = x_sq - 2.0 * xc + c_sq

    iota64 = lax.broadcasted_iota(I32, (N_DATA, N_CB), 1)
    rowmin = jnp.min(d2, axis=1, keepdims=True)
    labels = jnp.min(jnp.where(d2 == rowmin, iota64, N_CB), axis=1,
                     keepdims=True)
    labels_ref[...] = labels

    onehot = (iota64 == labels).astype(F32)

    r_i = lax.broadcasted_iota(I32, (CHUNK, CHUNK), 0)
    r_j = lax.broadcasted_iota(I32, (CHUNK, CHUNK), 1)
    tri = (r_j < r_i).astype(F32)
    run = jnp.zeros((1, N_CB), F32)
    ranks = []
    for c in range(N_DATA // CHUNK):
        oc = onehot[c * CHUNK:(c + 1) * CHUNK, :]
        cum_ex = lax.dot_general(tri, oc, (((1,), (0,)), ((), ())),
                                 preferred_element_type=F32) + run
        ranks.append(jnp.sum(cum_ex * oc, axis=1, keepdims=True))
        run = run + jnp.sum(oc, axis=0, keepdims=True)
    rank = jnp.concatenate(ranks, axis=0)
    counts = run

    u_i = lax.broadcasted_iota(I32, (N_CB, N_CB), 0)
    u_j = lax.broadcasted_iota(I32, (N_CB, N_CB), 1)
    upper = (u_i < u_j).astype(F32)
    offsets = lax.dot_general(counts, upper, (((1,), (0,)), ((), ())),
                              preferred_element_type=F32)

    posoff = lax.dot_general(onehot, offsets, (((1,), (1,)), ((), ())),
                             preferred_element_type=F32)
    pos_ref[...] = (rank + posoff).astype(I32)
    counts_ref[...] = counts.astype(I32)
    offsets_ref[...] = offsets.astype(I32)


def _run_prep(x, selector_centroids, interpret=False):
    return pl.pallas_call(
        _prep_body,
        out_shape=[
            jax.ShapeDtypeStruct((N_DATA, D_VECTOR), F32),
            jax.ShapeDtypeStruct((N_DATA, 1), I32),
            jax.ShapeDtypeStruct((N_DATA, 1), I32),
            jax.ShapeDtypeStruct((1, N_CB), I32),
            jax.ShapeDtypeStruct((1, N_CB), I32),
        ],
        interpret=interpret,
    )(x, selector_centroids)


def _adc_body(counts_sm, offsets_sm, xs_ref, cb_hbm, out_ref, cb_vmem, sem):
    b = pl.program_id(0)
    start = b * BLK
    xsb = xs_ref[...]
    q = lax.broadcasted_iota(I32, (BLK, 1), 0) + start

    out_ref[...] = jnp.zeros((N_SUB, BLK, N_CLUSTERS), F32)

    def visit(l, carry):
        o = offsets_sm[0, l]
        cnt = counts_sm[0, l]
        present = jnp.logical_and(
            cnt > 0,
            jnp.logical_and(o < start + BLK, o + cnt > start))

        @pl.when(present)
        def _():
            cp = pltpu.make_async_copy(cb_hbm.at[l], cb_vmem, sem)
            cp.start()
            cp.wait()
            mask = jnp.logical_and(q >= o, q < o + cnt)
            for m in range(N_SUB):
                xm = xsb[:, m * D_SUB:(m + 1) * D_SUB]
                cbm = cb_vmem[m]
                dot = lax.dot_general(xm, cbm, (((1,), (0,)), ((), ())),
                                      preferred_element_type=F32)
                sxq = jnp.sum(xm * xm, axis=1, keepdims=True)
                cbq = jnp.sum(cbm * cbm, axis=0, keepdims=True)
                sim = 2.0 * dot - sxq - cbq
                out_ref[m] = jnp.where(mask, sim, out_ref[m])

        return carry

    lax.fori_loop(0, N_CB, visit, 0)


def _run_adc(counts, offsets, xs, codebook, interpret=False):
    return pl.pallas_call(
        _adc_body,
        grid=(N_BLOCKS,),
        in_specs=[
            pl.BlockSpec(memory_space=pltpu.MemorySpace.SMEM),
            pl.BlockSpec(memory_space=pltpu.MemorySpace.SMEM),
            pl.BlockSpec((BLK, D_VECTOR), lambda b: (b, 0)),
            pl.BlockSpec(memory_space=pltpu.MemorySpace.HBM),
        ],
        out_shape=jax.ShapeDtypeStruct((N_SUB, N_DATA, N_CLUSTERS), F32),
        out_specs=pl.BlockSpec((N_SUB, BLK, N_CLUSTERS), lambda b: (0, b, 0)),
        scratch_shapes=[
            pltpu.VMEM((N_SUB, D_SUB, N_CLUSTERS), F32),
            pltpu.SemaphoreType.DMA,
        ],
        interpret=interpret,
    )(counts, offsets, xs, codebook)


def _sc_sort_rows(xt, pos2):
    info = plsc.get_sparse_core_info()
    nw = info.num_cores * info.num_subcores
    rpw = N_DATA // nw
    mesh = plsc.VectorSubcoreMesh(core_axis_name="c", subcore_axis_name="s")

    @functools.partial(
        pl.kernel,
        out_type=jax.ShapeDtypeStruct((N_DATA, D_VECTOR), F32),
        mesh=mesh,
        scratch_types=[
            pltpu.VMEM((1, rpw), I32),
            pltpu.VMEM((rpw, D_VECTOR), F32),
            pltpu.SemaphoreType.DMA,
        ],
    )
    def k(xt_hbm, pos_hbm, xs_hbm, idx_v, rows_v, sem):
        wid = lax.axis_index("s") * info.num_cores + lax.axis_index("c")
        pltpu.sync_copy(pos_hbm.at[wid], idx_v.at[0])
        pltpu.sync_copy(xt_hbm.at[pl.ds(wid * rpw, rpw)], rows_v)
        pltpu.async_copy(rows_v, xs_hbm.at[idx_v.at[0]], sem).wait()

    return k(xt, pos2)


def _sc_unsort_rows(sorted_flat, srcidx3):
    rows = N_SUB * N_DATA
    info = plsc.get_sparse_core_info()
    nw = info.num_cores * info.num_subcores
    rpw = rows // nw
    nch = rpw // 128
    mesh = plsc.VectorSubcoreMesh(core_axis_name="c", subcore_axis_name="s")

    @functools.partial(
        pl.kernel,
        out_type=jax.ShapeDtypeStruct((rows, N_CLUSTERS), F32),
        mesh=mesh,
        scratch_types=[
            pltpu.VMEM((1, 128), I32),
            pltpu.VMEM((128, N_CLUSTERS), F32),
            pltpu.SemaphoreType.DMA,
        ],
    )
    def k(src_hbm, idx_hbm, out_hbm, idx_v, rows_v, sem):
        wid = lax.axis_index("s") * info.num_cores + lax.axis_index("c")

        def chunk(j, carry):
            pltpu.sync_copy(idx_hbm.at[wid, j], idx_v.at[0])
            pltpu.async_copy(src_hbm.at[idx_v.at[0]], rows_v, sem).wait()
            pltpu.sync_copy(rows_v,
                            out_hbm.at[pl.ds(wid * rpw + j * 128, 128)])
            return carry

        lax.fori_loop(0, nch, chunk, 0)

    return k(sorted_flat, srcidx3)


def kernel(x, codebook, selector_centroids):
    info = plsc.get_sparse_core_info()
    nw = info.num_cores * info.num_subcores

    xt, labels2, pos2d, counts, offsets = _run_prep(x, selector_centroids)

    pos = pos2d.reshape(N_DATA)
    xs = _sc_sort_rows(xt, pos.reshape(nw, N_DATA // nw))

    out_sorted = _run_adc(counts, offsets, xs, codebook)

    srcidx = (jnp.arange(N_SUB, dtype=I32)[:, None] * N_DATA
              + pos[None, :]).reshape(nw, (N_SUB * N_DATA) // nw // 128, 128)
    out_flat = _sc_unsort_rows(
        out_sorted.reshape(N_SUB * N_DATA, N_CLUSTERS), srcidx)

    return out_flat.reshape(N_SUB, N_DATA, N_CLUSTERS), labels2.reshape(N_DATA)


# speedup vs baseline: 10.2795x; 1.4092x over previous
import functools

import jax
import jax.numpy as jnp
from jax import lax
from jax.experimental import pallas as pl
from jax.experimental.pallas import tpu as pltpu
from jax.experimental.pallas import tpu_sc as plsc

D_VECTOR = 256
N_DATA = 4096
N_SUB = 8
D_SUB = D_VECTOR // N_SUB
N_CLUSTERS = 256
N_CB = 64

BLK = 256
N_BLOCKS = N_DATA // BLK
CHUNK = 128

F32 = jnp.float32
I32 = jnp.int32


def _prep_body(x_ref, c_ref, xt_ref, labels_ref, pos_ref, counts_ref,
               offsets_ref, blk_cnt_ref, blk_list_ref):
    x = x_ref[...]
    xt = x.T
    xt_ref[...] = xt
    cent = c_ref[...]

    x_sq = jnp.sum(xt * xt, axis=1, keepdims=True)
    c_sq = jnp.sum(cent * cent, axis=0, keepdims=True)
    xc = lax.dot_general(xt, cent, (((1,), (0,)), ((), ())),
                         preferred_element_type=F32)
    d2 = x_sq - 2.0 * xc + c_sq

    iota64 = lax.broadcasted_iota(I32, (N_DATA, N_CB), 1)
    rowmin = jnp.min(d2, axis=1, keepdims=True)
    labels = jnp.min(jnp.where(d2 == rowmin, iota64, N_CB), axis=1,
                     keepdims=True)
    labels_ref[...] = labels

    onehot = (iota64 == labels).astype(F32)

    r_i = lax.broadcasted_iota(I32, (CHUNK, CHUNK), 0)
    r_j = lax.broadcasted_iota(I32, (CHUNK, CHUNK), 1)
    tri = (r_j < r_i).astype(F32)
    run = jnp.zeros((1, N_CB), F32)
    ranks = []
    for c in range(N_DATA // CHUNK):
        oc = onehot[c * CHUNK:(c + 1) * CHUNK, :]
        cum_ex = lax.dot_general(tri, oc, (((1,), (0,)), ((), ())),
                                 preferred_element_type=F32) + run
        ranks.append(jnp.sum(cum_ex * oc, axis=1, keepdims=True))
        run = run + jnp.sum(oc, axis=0, keepdims=True)
    rank = jnp.concatenate(ranks, axis=0)
    counts = run

    u_i = lax.broadcasted_iota(I32, (N_CB, N_CB), 0)
    u_j = lax.broadcasted_iota(I32, (N_CB, N_CB), 1)
    upper = (u_i < u_j).astype(F32)
    offsets = lax.dot_general(counts, upper, (((1,), (0,)), ((), ())),
                              preferred_element_type=F32)

    posoff = lax.dot_general(onehot, offsets, (((1,), (1,)), ((), ())),
                             preferred_element_type=F32)
    pos_ref[...] = (rank + posoff).astype(I32)
    counts_ref[...] = counts.astype(I32)
    offsets_ref[...] = offsets.astype(I32)

    bstart = lax.broadcasted_iota(I32, (N_BLOCKS, 1), 0).astype(F32) \
        * float(BLK)
    present = jnp.logical_and(
        counts > 0.5,
        jnp.logical_and(offsets < bstart + float(BLK),
                        offsets + counts > bstart)).astype(F32)
    blk_cnt_ref[...] = jnp.sum(present, axis=1, keepdims=True).astype(I32)
    rank_ex = lax.dot_general(present, upper, (((1,), (0,)), ((), ())),
                              preferred_element_type=F32)
    lab_row = lax.broadcasted_iota(I32, (N_BLOCKS, N_CB, 1), 1).astype(F32)
    sel = (rank_ex[:, :, None] ==
           lax.broadcasted_iota(I32, (N_BLOCKS, N_CB, N_CB), 2).astype(F32))
    blk_list_ref[...] = jnp.sum(
        jnp.where(sel, present[:, :, None] * lab_row, 0.0), axis=1
    ).astype(I32)


def _run_prep(x, selector_centroids, interpret=False):
    return pl.pallas_call(
        _prep_body,
        out_shape=[
            jax.ShapeDtypeStruct((N_DATA, D_VECTOR), F32),
            jax.ShapeDtypeStruct((N_DATA, 1), I32),
            jax.ShapeDtypeStruct((N_DATA, 1), I32),
            jax.ShapeDtypeStruct((1, N_CB), I32),
            jax.ShapeDtypeStruct((1, N_CB), I32),
            jax.ShapeDtypeStruct((N_BLOCKS, 1), I32),
            jax.ShapeDtypeStruct((N_BLOCKS, N_CB), I32),
        ],
        interpret=interpret,
    )(x, selector_centroids)


def _adc_body(counts_sm, offsets_sm, blk_cnt_sm, blk_list_sm, xs_ref, cb_hbm,
              out_ref, cb_vmem, sem):
    b = pl.program_id(0)
    start = b * BLK
    xsb = xs_ref[...]
    q = lax.broadcasted_iota(I32, (BLK, 1), 0) + start
    nb = blk_cnt_sm[0, 0, 0]

    def dma(j, slot):
        return pltpu.make_async_copy(cb_hbm.at[blk_list_sm[0, 0, j]],
                                     cb_vmem.at[slot], sem.at[slot])

    dma(0, 0).start()

    def visit(j, carry):
        slot = lax.rem(j, 2)

        @pl.when(j + 1 < nb)
        def _():
            dma(j + 1, 1 - slot).start()

        dma(j, slot).wait()
        l = blk_list_sm[0, 0, j]
        o = offsets_sm[0, l]
        cnt = counts_sm[0, l]
        mask = jnp.logical_and(q >= o, q < o + cnt)
        for m in range(N_SUB):
            xm = xsb[:, m * D_SUB:(m + 1) * D_SUB]
            cbm = cb_vmem[slot, m]
            dot = lax.dot_general(xm, cbm, (((1,), (0,)), ((), ())),
                                  preferred_element_type=F32)
            sxq = jnp.sum(xm * xm, axis=1, keepdims=True)
            cbq = jnp.sum(cbm * cbm, axis=0, keepdims=True)
            sim = 2.0 * dot - sxq - cbq
            out_ref[m] = jnp.where(mask, sim, out_ref[m])
        return carry

    lax.fori_loop(0, nb, visit, 0)


def _run_adc(counts, offsets, blk_cnt, blk_list, xs, codebook,
             interpret=False):
    return pl.pallas_call(
        _adc_body,
        grid=(N_BLOCKS,),
        in_specs=[
            pl.BlockSpec(memory_space=pltpu.MemorySpace.SMEM),
            pl.BlockSpec(memory_space=pltpu.MemorySpace.SMEM),
            pl.BlockSpec((1, 1, 1), lambda b: (b, 0, 0),
                         memory_space=pltpu.MemorySpace.SMEM),
            pl.BlockSpec((1, 1, N_CB), lambda b: (b, 0, 0),
                         memory_space=pltpu.MemorySpace.SMEM),
            pl.BlockSpec((BLK, D_VECTOR), lambda b: (b, 0)),
            pl.BlockSpec(memory_space=pltpu.MemorySpace.HBM),
        ],
        out_shape=jax.ShapeDtypeStruct((N_SUB, N_DATA, N_CLUSTERS), F32),
        out_specs=pl.BlockSpec((N_SUB, BLK, N_CLUSTERS), lambda b: (0, b, 0)),
        scratch_shapes=[
            pltpu.VMEM((2, N_SUB, D_SUB, N_CLUSTERS), F32),
            pltpu.SemaphoreType.DMA((2,)),
        ],
        interpret=interpret,
    )(counts, offsets, blk_cnt, blk_list, xs, codebook)


def _sc_sort_rows(xt, pos2):
    info = plsc.get_sparse_core_info()
    nw = info.num_cores * info.num_subcores
    rpw = N_DATA // nw
    mesh = plsc.VectorSubcoreMesh(core_axis_name="c", subcore_axis_name="s")

    @functools.partial(
        pl.kernel,
        out_type=jax.ShapeDtypeStruct((N_DATA, D_VECTOR), F32),
        mesh=mesh,
        scratch_types=[
            pltpu.VMEM((1, rpw), I32),
            pltpu.VMEM((rpw, D_VECTOR), F32),
            pltpu.SemaphoreType.DMA,
        ],
    )
    def k(xt_hbm, pos_hbm, xs_hbm, idx_v, rows_v, sem):
        wid = lax.axis_index("s") * info.num_cores + lax.axis_index("c")
        pltpu.sync_copy(pos_hbm.at[wid], idx_v.at[0])
        pltpu.sync_copy(xt_hbm.at[pl.ds(wid * rpw, rpw)], rows_v)
        pltpu.async_copy(rows_v, xs_hbm.at[idx_v.at[0]], sem).wait()

    return k(xt, pos2)


def _sc_unsort_rows(sorted_flat, srcidx3):
    rows = N_SUB * N_DATA
    info = plsc.get_sparse_core_info()
    nw = info.num_cores * info.num_subcores
    rpw = rows // nw
    nch = rpw // 128
    mesh = plsc.VectorSubcoreMesh(core_axis_name="c", subcore_axis_name="s")

    @functools.partial(
        pl.kernel,
        out_type=jax.ShapeDtypeStruct((rows, N_CLUSTERS), F32),
        mesh=mesh,
        scratch_types=[
            pltpu.VMEM((1, 128), I32),
            pltpu.VMEM((128, N_CLUSTERS), F32),
            pltpu.SemaphoreType.DMA,
        ],
    )
    def k(src_hbm, idx_hbm, out_hbm, idx_v, rows_v, sem):
        wid = lax.axis_index("s") * info.num_cores + lax.axis_index("c")

        def chunk(j, carry):
            pltpu.sync_copy(idx_hbm.at[wid, j], idx_v.at[0])
            pltpu.async_copy(src_hbm.at[idx_v.at[0]], rows_v, sem).wait()
            pltpu.sync_copy(rows_v,
                            out_hbm.at[pl.ds(wid * rpw + j * 128, 128)])
            return carry

        lax.fori_loop(0, nch, chunk, 0)

    return k(sorted_flat, srcidx3)


def kernel(x, codebook, selector_centroids):
    info = plsc.get_sparse_core_info()
    nw = info.num_cores * info.num_subcores

    (xt, labels2, pos2d, counts, offsets, blk_cnt,
     blk_list) = _run_prep(x, selector_centroids)

    pos = pos2d.reshape(N_DATA)
    xs = _sc_sort_rows(xt, pos.reshape(nw, N_DATA // nw))

    out_sorted = _run_adc(counts, offsets,
                          blk_cnt.reshape(N_BLOCKS, 1, 1),
                          blk_list.reshape(N_BLOCKS, 1, N_CB), xs, codebook)

    srcidx = (jnp.arange(N_SUB, dtype=I32)[:, None] * N_DATA
              + pos[None, :]).reshape(nw, (N_SUB * N_DATA) // nw // 128, 128)
    out_flat = _sc_unsort_rows(
        out_sorted.reshape(N_SUB * N_DATA, N_CLUSTERS), srcidx)

    return out_flat.reshape(N_SUB, N_DATA, N_CLUSTERS), labels2.reshape(N_DATA)


# speedup vs baseline: 10.2883x; 1.0009x over previous
import functools

import jax
import jax.numpy as jnp
from jax import lax
from jax.experimental import pallas as pl
from jax.experimental.pallas import tpu as pltpu
from jax.experimental.pallas import tpu_sc as plsc

D_VECTOR = 256
N_DATA = 4096
N_SUB = 8
D_SUB = D_VECTOR // N_SUB
N_CLUSTERS = 256
N_CB = 64

BLK = 256
N_BLOCKS = N_DATA // BLK
CHUNK = 128

F32 = jnp.float32
I32 = jnp.int32


def _prep_body(x_ref, c_ref, xt_ref, labels_ref, pos_ref, counts_ref,
               offsets_ref, blk_cnt_ref, blk_list_ref):
    x = x_ref[...]
    xt = x.T
    xt_ref[...] = xt
    cent = c_ref[...]

    x_sq = jnp.sum(xt * xt, axis=1, keepdims=True)
    c_sq = jnp.sum(cent * cent, axis=0, keepdims=True)
    xc = lax.dot_general(x, cent, (((0,), (0,)), ((), ())),
                         preferred_element_type=F32)
    d2 = x_sq - 2.0 * xc + c_sq

    iota64 = lax.broadcasted_iota(I32, (N_DATA, N_CB), 1)
    rowmin = jnp.min(d2, axis=1, keepdims=True)
    labels = jnp.min(jnp.where(d2 == rowmin, iota64, N_CB), axis=1,
                     keepdims=True)
    labels_ref[...] = labels

    onehot = (iota64 == labels).astype(F32)

    r_i = lax.broadcasted_iota(I32, (CHUNK, CHUNK), 0)
    r_j = lax.broadcasted_iota(I32, (CHUNK, CHUNK), 1)
    tri = (r_j < r_i).astype(F32)
    run = jnp.zeros((1, N_CB), F32)
    ranks = []
    for c in range(N_DATA // CHUNK):
        oc = onehot[c * CHUNK:(c + 1) * CHUNK, :]
        cum_ex = lax.dot_general(tri, oc, (((1,), (0,)), ((), ())),
                                 preferred_element_type=F32) + run
        ranks.append(jnp.sum(cum_ex * oc, axis=1, keepdims=True))
        run = run + jnp.sum(oc, axis=0, keepdims=True)
    rank = jnp.concatenate(ranks, axis=0)
    counts = run

    u_i = lax.broadcasted_iota(I32, (N_CB, N_CB), 0)
    u_j = lax.broadcasted_iota(I32, (N_CB, N_CB), 1)
    upper = (u_i < u_j).astype(F32)
    offsets = lax.dot_general(counts, upper, (((1,), (0,)), ((), ())),
                              preferred_element_type=F32)

    posoff = lax.dot_general(onehot, offsets, (((1,), (1,)), ((), ())),
                             preferred_element_type=F32)
    pos_ref[...] = (rank + posoff).astype(I32)
    counts_ref[...] = counts.astype(I32)
    offsets_ref[...] = offsets.astype(I32)

    bstart = lax.broadcasted_iota(I32, (N_BLOCKS, 1), 0).astype(F32) \
        * float(BLK)
    present = jnp.logical_and(
        counts > 0.5,
        jnp.logical_and(offsets < bstart + float(BLK),
                        offsets + counts > bstart)).astype(F32)
    blk_cnt_ref[...] = jnp.sum(present, axis=1, keepdims=True).astype(I32)
    rank_ex = lax.dot_general(present, upper, (((1,), (0,)), ((), ())),
                              preferred_element_type=F32)
    lab_row = lax.broadcasted_iota(I32, (N_BLOCKS, N_CB, 1), 1).astype(F32)
    sel = (rank_ex[:, :, None] ==
           lax.broadcasted_iota(I32, (N_BLOCKS, N_CB, N_CB), 2).astype(F32))
    blk_list_ref[...] = jnp.sum(
        jnp.where(sel, present[:, :, None] * lab_row, 0.0), axis=1
    ).astype(I32)


def _run_prep(x, selector_centroids, interpret=False):
    return pl.pallas_call(
        _prep_body,
        out_shape=[
            jax.ShapeDtypeStruct((N_DATA, D_VECTOR), F32),
            jax.ShapeDtypeStruct((N_DATA, 1), I32),
            jax.ShapeDtypeStruct((N_DATA, 1), I32),
            jax.ShapeDtypeStruct((1, N_CB), I32),
            jax.ShapeDtypeStruct((1, N_CB), I32),
            jax.ShapeDtypeStruct((N_BLOCKS, 1), I32),
            jax.ShapeDtypeStruct((N_BLOCKS, N_CB), I32),
        ],
        interpret=interpret,
    )(x, selector_centroids)


def _adc_body(counts_sm, offsets_sm, blk_cnt_sm, blk_list_sm, xs_ref, cb_hbm,
              out_ref, cb_vmem, sem):
    b = pl.program_id(0)
    start = b * BLK
    xsb = xs_ref[...]
    q = lax.broadcasted_iota(I32, (BLK, 1), 0) + start
    nb = blk_cnt_sm[0, 0, 0]

    def dma(j, slot):
        return pltpu.make_async_copy(cb_hbm.at[blk_list_sm[0, 0, j]],
                                     cb_vmem.at[slot], sem.at[slot])

    dma(0, 0).start()

    def visit(j, carry):
        slot = lax.rem(j, 2)

        @pl.when(j + 1 < nb)
        def _():
            dma(j + 1, 1 - slot).start()

        dma(j, slot).wait()
        l = blk_list_sm[0, 0, j]
        o = offsets_sm[0, l]
        cnt = counts_sm[0, l]
        mask = jnp.logical_and(q >= o, q < o + cnt)
        for m in range(N_SUB):
            xm = xsb[:, m * D_SUB:(m + 1) * D_SUB]
            cbm = cb_vmem[slot, m]
            dot = lax.dot_general(xm, cbm, (((1,), (0,)), ((), ())),
                                  preferred_element_type=F32)
            sxq = jnp.sum(xm * xm, axis=1, keepdims=True)
            cbq = jnp.sum(cbm * cbm, axis=0, keepdims=True)
            sim = 2.0 * dot - sxq - cbq
            out_ref[m] = jnp.where(mask, sim, out_ref[m])
        return carry

    lax.fori_loop(0, nb, visit, 0)


def _run_adc(counts, offsets, blk_cnt, blk_list, xs, codebook,
             interpret=False):
    return pl.pallas_call(
        _adc_body,
        grid=(N_BLOCKS,),
        in_specs=[
            pl.BlockSpec(memory_space=pltpu.MemorySpace.SMEM),
            pl.BlockSpec(memory_space=pltpu.MemorySpace.SMEM),
            pl.BlockSpec((1, 1, 1), lambda b: (b, 0, 0),
                         memory_space=pltpu.MemorySpace.SMEM),
            pl.BlockSpec((1, 1, N_CB), lambda b: (b, 0, 0),
                         memory_space=pltpu.MemorySpace.SMEM),
            pl.BlockSpec((BLK, D_VECTOR), lambda b: (b, 0)),
            pl.BlockSpec(memory_space=pltpu.MemorySpace.HBM),
        ],
        out_shape=jax.ShapeDtypeStruct((N_SUB, N_DATA, N_CLUSTERS), F32),
        out_specs=pl.BlockSpec((N_SUB, BLK, N_CLUSTERS), lambda b: (0, b, 0)),
        scratch_shapes=[
            pltpu.VMEM((2, N_SUB, D_SUB, N_CLUSTERS), F32),
            pltpu.SemaphoreType.DMA((2,)),
        ],
        interpret=interpret,
    )(counts, offsets, blk_cnt, blk_list, xs, codebook)


def _sc_sort_rows(xt, pos2):
    info = plsc.get_sparse_core_info()
    nw = info.num_cores * info.num_subcores
    rpw = N_DATA // nw
    mesh = plsc.VectorSubcoreMesh(core_axis_name="c", subcore_axis_name="s")

    @functools.partial(
        pl.kernel,
        out_type=jax.ShapeDtypeStruct((N_DATA, D_VECTOR), F32),
        mesh=mesh,
        scratch_types=[
            pltpu.VMEM((1, rpw), I32),
            pltpu.VMEM((rpw, D_VECTOR), F32),
            pltpu.SemaphoreType.DMA,
        ],
    )
    def k(xt_hbm, pos_hbm, xs_hbm, idx_v, rows_v, sem):
        wid = lax.axis_index("s") * info.num_cores + lax.axis_index("c")
        pltpu.sync_copy(pos_hbm.at[wid], idx_v.at[0])
        pltpu.sync_copy(xt_hbm.at[pl.ds(wid * rpw, rpw)], rows_v)
        pltpu.async_copy(rows_v, xs_hbm.at[idx_v.at[0]], sem).wait()

    return k(xt, pos2)


def _sc_unsort_rows(sorted_flat, srcidx3):
    rows = N_SUB * N_DATA
    info = plsc.get_sparse_core_info()
    nw = info.num_cores * info.num_subcores
    rpw = rows // nw
    nch = rpw // 128
    mesh = plsc.VectorSubcoreMesh(core_axis_name="c", subcore_axis_name="s")

    @functools.partial(
        pl.kernel,
        out_type=jax.ShapeDtypeStruct((rows, N_CLUSTERS), F32),
        mesh=mesh,
        scratch_types=[
            pltpu.VMEM((1, 128), I32),
            pltpu.VMEM((128, N_CLUSTERS), F32),
            pltpu.SemaphoreType.DMA,
        ],
    )
    def k(src_hbm, idx_hbm, out_hbm, idx_v, rows_v, sem):
        wid = lax.axis_index("s") * info.num_cores + lax.axis_index("c")

        def chunk(j, carry):
            pltpu.sync_copy(idx_hbm.at[wid, j], idx_v.at[0])
            pltpu.async_copy(src_hbm.at[idx_v.at[0]], rows_v, sem).wait()
            pltpu.sync_copy(rows_v,
                            out_hbm.at[pl.ds(wid * rpw + j * 128, 128)])
            return carry

        lax.fori_loop(0, nch, chunk, 0)

    return k(sorted_flat, srcidx3)


def kernel(x, codebook, selector_centroids):
    info = plsc.get_sparse_core_info()
    nw = info.num_cores * info.num_subcores

    (xt, labels2, pos2d, counts, offsets, blk_cnt,
     blk_list) = _run_prep(x, selector_centroids)

    pos = pos2d.reshape(N_DATA)
    xs = _sc_sort_rows(xt, pos.reshape(nw, N_DATA // nw))

    out_sorted = _run_adc(counts, offsets,
                          blk_cnt.reshape(N_BLOCKS, 1, 1),
                          blk_list.reshape(N_BLOCKS, 1, N_CB), xs, codebook)

    srcidx = (jnp.arange(N_SUB, dtype=I32)[:, None] * N_DATA
              + pos[None, :]).reshape(nw, (N_SUB * N_DATA) // nw // 128, 128)
    out_flat = _sc_unsort_rows(
        out_sorted.reshape(N_SUB * N_DATA, N_CLUSTERS), srcidx)

    return out_flat.reshape(N_SUB, N_DATA, N_CLUSTERS), labels2.reshape(N_DATA)
